# baseline (speedup 1.0000x reference)
import functools

import jax
import jax.numpy as jnp
from jax import lax
from jax.experimental import pallas as pl
from jax.experimental.pallas import tpu as pltpu
from jax.experimental.pallas import tpu_sc as plsc

N_NODES = 50000
N_EDGES = 800000
N_PAIRS = N_EDGES // 2
D_NODE = 31
D_HID = 64
BOX = 6.0

NUM_CORES = 2
NUM_SUBCORES = 16
NW = NUM_CORES * NUM_SUBCORES
CHUNK = 400
TOTAL_CHUNKS = N_PAIRS // CHUNK
K_STEPS = -(-TOTAL_CHUNKS // NW)
TW = D_NODE + 1

BEH = 8000
TC_GRID = N_PAIRS // BEH


def _sc_gather_body(t_hbm, rs_hbm, rr_hbm, g_hbm,
                    idx_se, idx_so, idx_re, idx_ro, sve, svo, rve, rvo,
                    sem):
  wid = lax.axis_index("s") * NUM_CORES + lax.axis_index("c")

  def step(k, _):
    q = k * NW + wid

    @pl.when(q < TOTAL_CHUNKS)
    def _():
      base = pl.multiple_of(q * CHUNK, 8)
      pltpu.sync_copy(rs_hbm.at[pl.ds(base, CHUNK)], idx_se)
      pltpu.sync_copy(rs_hbm.at[pl.ds(base + N_PAIRS, CHUNK)], idx_so)
      pltpu.sync_copy(rr_hbm.at[pl.ds(base, CHUNK)], idx_re)
      pltpu.sync_copy(rr_hbm.at[pl.ds(base + N_PAIRS, CHUNK)], idx_ro)
      c1 = pltpu.async_copy(t_hbm.at[idx_se], sve, sem)
      c2 = pltpu.async_copy(t_hbm.at[idx_so], svo, sem)
      c3 = pltpu.async_copy(t_hbm.at[idx_re], rve, sem)
      c4 = pltpu.async_copy(t_hbm.at[idx_ro], rvo, sem)
      c1.wait()
      c2.wait()
      c3.wait()
      c4.wait()
      pltpu.sync_copy(sve, g_hbm.at[pl.ds(base, CHUNK), pl.ds(0, TW)])
      pltpu.sync_copy(rve, g_hbm.at[pl.ds(base, CHUNK), pl.ds(TW, TW)])
      pltpu.sync_copy(svo, g_hbm.at[pl.ds(base, CHUNK), pl.ds(2 * TW, TW)])
      pltpu.sync_copy(rvo, g_hbm.at[pl.ds(base, CHUNK), pl.ds(3 * TW, TW)])

    return 0

  lax.fori_loop(0, K_STEPS, step, 0)


_sc_gather = functools.partial(
    pl.kernel,
    out_type=jax.ShapeDtypeStruct((N_PAIRS, 128), jnp.float32),
    mesh=plsc.VectorSubcoreMesh(core_axis_name="c", subcore_axis_name="s",
                                num_cores=NUM_CORES,
                                num_subcores=NUM_SUBCORES),
    scratch_types=[
        pltpu.VMEM((CHUNK,), jnp.int32),
        pltpu.VMEM((CHUNK,), jnp.int32),
        pltpu.VMEM((CHUNK,), jnp.int32),
        pltpu.VMEM((CHUNK,), jnp.int32),
        pltpu.VMEM((CHUNK, TW), jnp.float32),
        pltpu.VMEM((CHUNK, TW), jnp.float32),
        pltpu.VMEM((CHUNK, TW), jnp.float32),
        pltpu.VMEM((CHUNK, TW), jnp.float32),
        pltpu.SemaphoreType.DMA,
    ],
    compiler_params=pltpu.CompilerParams(use_tc_tiling_on_sc=False),
)(_sc_gather_body)


def _unpack_pos(col):
  hi = jnp.uint32(0xFFFF0000)
  u = lax.bitcast_convert_type(col, jnp.uint32)
  px = lax.bitcast_convert_type(u & hi, jnp.float32)
  py = lax.bitcast_convert_type(u << 16, jnp.float32)
  return px, py


def _wrap(d):
  half = BOX / 2
  d = jnp.where(d > half, d - BOX, d)
  return jnp.where(d <= -half, d + BOX, d)


def _tc_mlp_body(g_ref, m_ref, b1_ref, w2_ref, b2_ref, o_ref):
  g = g_ref[...]
  sxe, sye = _unpack_pos(g[:, TW - 1:TW])
  rxe, rye = _unpack_pos(g[:, 2 * TW - 1:2 * TW])
  sxo, syo = _unpack_pos(g[:, 3 * TW - 1:3 * TW])
  rxo, ryo = _unpack_pos(g[:, 4 * TW - 1:4 * TW])
  d0e = _wrap(sxe - rxe)
  d1e = _wrap(sye - rye)
  d0o = _wrap(sxo - rxo)
  d1o = _wrap(syo - ryo)
  lane = lax.broadcasted_iota(jnp.int32, (BEH, 128), 1)
  g = jnp.where(lane == TW - 1, d0e, g)
  g = jnp.where(lane == 2 * TW - 1, d1e, g)
  g = jnp.where(lane == 3 * TW - 1, d0o, g)
  g = jnp.where(lane == 4 * TW - 1, d1o, g)
  h = jnp.dot(g, m_ref[...], preferred_element_type=jnp.float32)
  h = jnp.maximum(h + b1_ref[...], 0.0)
  o = jnp.dot(h, w2_ref[...], preferred_element_type=jnp.float32)
  o = jnp.maximum(o + b2_ref[...], 0.0)
  o_ref[0] = o[:, 0:D_HID]
  o_ref[1] = o[:, D_HID:2 * D_HID]


def _tc_mlp(g, m, b1r, w2, b2r):
  return pl.pallas_call(
      _tc_mlp_body,
      grid=(TC_GRID,),
      in_specs=[
          pl.BlockSpec((BEH, 128), lambda i: (i, 0)),
          pl.BlockSpec((128, 128), lambda i: (0, 0)),
          pl.BlockSpec((1, 128), lambda i: (0, 0)),
          pl.BlockSpec((128, 128), lambda i: (0, 0)),
          pl.BlockSpec((1, 128), lambda i: (0, 0)),
      ],
      out_specs=pl.BlockSpec((2, BEH, D_HID), lambda i: (0, i, 0)),
      out_shape=jax.ShapeDtypeStruct((2, N_PAIRS, D_HID), jnp.float32),
  )(g, m, b1r, w2, b2r)


@jax.jit
def kernel(V_no_pos, V_pos, R_s, R_r, W1, b1, W2, b2):
  v = V_no_pos[0]
  p = V_pos[0]
  px_bits = lax.bitcast_convert_type(
      p[:, 0].astype(jnp.bfloat16), jnp.uint16).astype(jnp.uint32)
  py_bits = lax.bitcast_convert_type(
      p[:, 1].astype(jnp.bfloat16), jnp.uint16).astype(jnp.uint32)
  packed = lax.bitcast_convert_type((px_bits << 16) | py_bits, jnp.float32)
  table = jnp.concatenate([v, packed[:, None]], axis=1)

  rs = R_s.reshape(N_EDGES).astype(jnp.int32)
  rr = R_r.reshape(N_EDGES).astype(jnp.int32)
  g = _sc_gather(table, rs, rr)

  mhalf = jnp.concatenate(
      [W1[:, 0:D_NODE].T,
       W1[:, 2 * D_NODE:2 * D_NODE + 1].T,
       W1[:, D_NODE:2 * D_NODE].T,
       W1[:, 2 * D_NODE + 1:2 * D_NODE + 2].T],
      axis=0)
  z = jnp.zeros((64, 64), jnp.float32)
  m = jnp.concatenate(
      [jnp.concatenate([mhalf, z], axis=1),
       jnp.concatenate([z, mhalf], axis=1)], axis=0)
  w2 = jnp.concatenate(
      [jnp.concatenate([W2.T, z], axis=1),
       jnp.concatenate([z, W2.T], axis=1)], axis=0)
  b1r = jnp.concatenate([b1, b1])[None, :]
  b2r = jnp.concatenate([b2, b2])[None, :]

  out = _tc_mlp(g, m, b1r, w2, b2r)
  return out.reshape(1, N_EDGES, D_HID)


# speedup vs baseline: 1.0030x; 1.0030x over previous
import functools

import jax
import jax.numpy as jnp
from jax import lax
from jax.experimental import pallas as pl
from jax.experimental.pallas import tpu as pltpu
from jax.experimental.pallas import tpu_sc as plsc

N_NODES = 50000
N_EDGES = 800000
N_PAIRS = N_EDGES // 2
D_NODE = 31
D_HID = 64
BOX = 6.0

NUM_CORES = 2
NUM_SUBCORES = 16
NW = NUM_CORES * NUM_SUBCORES
CHUNK = 400
HP = N_PAIRS // 2
TOTAL_CHUNKS = HP // CHUNK
K_STEPS = -(-TOTAL_CHUNKS // NW)
TW = D_NODE + 1

BEH = 4000
TC_GRID = HP // BEH


def _make_sc_body(eoff):
  def _sc_gather_body(t_hbm, rs_hbm, rr_hbm, g_hbm,
                      idx_se, idx_so, idx_re, idx_ro, sve, svo, rve, rvo,
                      sem):
    wid = lax.axis_index("s") * NUM_CORES + lax.axis_index("c")

    def step(k, _):
      q = k * NW + wid

      @pl.when(q < TOTAL_CHUNKS)
      def _():
        base = pl.multiple_of(q * CHUNK, 8)
        pltpu.sync_copy(rs_hbm.at[pl.ds(base + eoff, CHUNK)], idx_se)
        pltpu.sync_copy(rs_hbm.at[pl.ds(base + eoff + HP, CHUNK)], idx_so)
        pltpu.sync_copy(rr_hbm.at[pl.ds(base + eoff, CHUNK)], idx_re)
        pltpu.sync_copy(rr_hbm.at[pl.ds(base + eoff + HP, CHUNK)], idx_ro)
        c1 = pltpu.async_copy(t_hbm.at[idx_se], sve, sem)
        c2 = pltpu.async_copy(t_hbm.at[idx_so], svo, sem)
        c3 = pltpu.async_copy(t_hbm.at[idx_re], rve, sem)
        c4 = pltpu.async_copy(t_hbm.at[idx_ro], rvo, sem)
        c1.wait()
        c2.wait()
        c3.wait()
        c4.wait()
        pltpu.sync_copy(sve, g_hbm.at[pl.ds(base, CHUNK), pl.ds(0, TW)])
        pltpu.sync_copy(rve, g_hbm.at[pl.ds(base, CHUNK), pl.ds(TW, TW)])
        pltpu.sync_copy(svo, g_hbm.at[pl.ds(base, CHUNK), pl.ds(2 * TW, TW)])
        pltpu.sync_copy(rvo, g_hbm.at[pl.ds(base, CHUNK), pl.ds(3 * TW, TW)])

      return 0

    lax.fori_loop(0, K_STEPS, step, 0)

  return _sc_gather_body


def _make_sc_gather(eoff):
  return functools.partial(
      pl.kernel,
      out_type=jax.ShapeDtypeStruct((HP, 128), jnp.float32),
      mesh=plsc.VectorSubcoreMesh(core_axis_name="c", subcore_axis_name="s",
                                  num_cores=NUM_CORES,
                                  num_subcores=NUM_SUBCORES),
      scratch_types=[
          pltpu.VMEM((CHUNK,), jnp.int32),
          pltpu.VMEM((CHUNK,), jnp.int32),
          pltpu.VMEM((CHUNK,), jnp.int32),
          pltpu.VMEM((CHUNK,), jnp.int32),
          pltpu.VMEM((CHUNK, TW), jnp.float32),
          pltpu.VMEM((CHUNK, TW), jnp.float32),
          pltpu.VMEM((CHUNK, TW), jnp.float32),
          pltpu.VMEM((CHUNK, TW), jnp.float32),
          pltpu.SemaphoreType.DMA,
      ],
      compiler_params=pltpu.CompilerParams(use_tc_tiling_on_sc=False),
  )(_make_sc_body(eoff))


_sc_gather_0 = _make_sc_gather(0)
_sc_gather_1 = _make_sc_gather(2 * HP)


def _unpack_pos(col):
  hi = jnp.uint32(0xFFFF0000)
  u = lax.bitcast_convert_type(col, jnp.uint32)
  px = lax.bitcast_convert_type(u & hi, jnp.float32)
  py = lax.bitcast_convert_type(u << 16, jnp.float32)
  return px, py


def _wrap(d):
  half = BOX / 2
  d = jnp.where(d > half, d - BOX, d)
  return jnp.where(d <= -half, d + BOX, d)


def _tc_mlp_body(g_ref, m_ref, b1_ref, w2_ref, b2_ref, o_ref):
  g = g_ref[...]
  sxe, sye = _unpack_pos(g[:, TW - 1:TW])
  rxe, rye = _unpack_pos(g[:, 2 * TW - 1:2 * TW])
  sxo, syo = _unpack_pos(g[:, 3 * TW - 1:3 * TW])
  rxo, ryo = _unpack_pos(g[:, 4 * TW - 1:4 * TW])
  d0e = _wrap(sxe - rxe)
  d1e = _wrap(sye - rye)
  d0o = _wrap(sxo - rxo)
  d1o = _wrap(syo - ryo)
  lane = lax.broadcasted_iota(jnp.int32, (BEH, 128), 1)
  g = jnp.where(lane == TW - 1, d0e, g)
  g = jnp.where(lane == 2 * TW - 1, d1e, g)
  g = jnp.where(lane == 3 * TW - 1, d0o, g)
  g = jnp.where(lane == 4 * TW - 1, d1o, g)
  h = jnp.dot(g, m_ref[...], preferred_element_type=jnp.float32)
  h = jnp.maximum(h + b1_ref[...], 0.0)
  o = jnp.dot(h, w2_ref[...], preferred_element_type=jnp.float32)
  o = jnp.maximum(o + b2_ref[...], 0.0)
  o_ref[0] = o[:, 0:D_HID]
  o_ref[1] = o[:, D_HID:2 * D_HID]


def _tc_mlp(g, m, b1r, w2, b2r):
  return pl.pallas_call(
      _tc_mlp_body,
      grid=(TC_GRID,),
      in_specs=[
          pl.BlockSpec((BEH, 128), lambda i: (i, 0)),
          pl.BlockSpec((128, 128), lambda i: (0, 0)),
          pl.BlockSpec((1, 128), lambda i: (0, 0)),
          pl.BlockSpec((128, 128), lambda i: (0, 0)),
          pl.BlockSpec((1, 128), lambda i: (0, 0)),
      ],
      out_specs=pl.BlockSpec((2, BEH, D_HID), lambda i: (0, i, 0)),
      out_shape=jax.ShapeDtypeStruct((2, HP, D_HID), jnp.float32),
  )(g, m, b1r, w2, b2r)


@jax.jit
def kernel(V_no_pos, V_pos, R_s, R_r, W1, b1, W2, b2):
  v = V_no_pos[0]
  p = V_pos[0]
  px_bits = lax.bitcast_convert_type(
      p[:, 0].astype(jnp.bfloat16), jnp.uint16).astype(jnp.uint32)
  py_bits = lax.bitcast_convert_type(
      p[:, 1].astype(jnp.bfloat16), jnp.uint16).astype(jnp.uint32)
  packed = lax.bitcast_convert_type((px_bits << 16) | py_bits, jnp.float32)
  table = jnp.concatenate([v, packed[:, None]], axis=1)

  rs = R_s.reshape(N_EDGES).astype(jnp.int32)
  rr = R_r.reshape(N_EDGES).astype(jnp.int32)
  g0 = _sc_gather_0(table, rs, rr)
  g1 = _sc_gather_1(table, rs, rr)

  mhalf = jnp.concatenate(
      [W1[:, 0:D_NODE].T,
       W1[:, 2 * D_NODE:2 * D_NODE + 1].T,
       W1[:, D_NODE:2 * D_NODE].T,
       W1[:, 2 * D_NODE + 1:2 * D_NODE + 2].T],
      axis=0)
  z = jnp.zeros((64, 64), jnp.float32)
  m = jnp.concatenate(
      [jnp.concatenate([mhalf, z], axis=1),
       jnp.concatenate([z, mhalf], axis=1)], axis=0)
  w2 = jnp.concatenate(
      [jnp.concatenate([W2.T, z], axis=1),
       jnp.concatenate([z, W2.T], axis=1)], axis=0)
  b1r = jnp.concatenate([b1, b1])[None, :]
  b2r = jnp.concatenate([b2, b2])[None, :]

  o0 = _tc_mlp(g0, m, b1r, w2, b2r)
  o1 = _tc_mlp(g1, m, b1r, w2, b2r)
  out = jnp.concatenate(
      [o0.reshape(2 * HP, D_HID), o1.reshape(2 * HP, D_HID)], axis=0)
  return out[None]


# speedup vs baseline: 1.0060x; 1.0030x over previous
import functools

import jax
import jax.numpy as jnp
from jax import lax
from jax.experimental import pallas as pl
from jax.experimental.pallas import tpu as pltpu
from jax.experimental.pallas import tpu_sc as plsc

N_NODES = 50000
N_EDGES = 800000
N_PAIRS = N_EDGES // 2
D_NODE = 31
D_HID = 64
BOX = 6.0

NUM_CORES = 2
NUM_SUBCORES = 16
NW = NUM_CORES * NUM_SUBCORES
CHUNK = 400
TOTAL_CHUNKS = N_PAIRS // CHUNK
K_STEPS = -(-TOTAL_CHUNKS // NW)
TW = D_NODE + 1

BEH = 4000
TC_GRID = N_PAIRS // BEH


def _sc_gather_body(t_hbm, rs_hbm, rr_hbm, g_hbm,
                    idx_se, idx_so, idx_re, idx_ro, sve, svo, rve, rvo,
                    sem):
  wid = lax.axis_index("s") * NUM_CORES + lax.axis_index("c")

  def step(k, _):
    q = k * NW + wid

    @pl.when(q < TOTAL_CHUNKS)
    def _():
      base = pl.multiple_of(q * CHUNK, 8)
      pltpu.sync_copy(rs_hbm.at[pl.ds(base, CHUNK)], idx_se)
      pltpu.sync_copy(rs_hbm.at[pl.ds(base + N_PAIRS, CHUNK)], idx_so)
      pltpu.sync_copy(rr_hbm.at[pl.ds(base, CHUNK)], idx_re)
      pltpu.sync_copy(rr_hbm.at[pl.ds(base + N_PAIRS, CHUNK)], idx_ro)
      c1 = pltpu.async_copy(t_hbm.at[idx_se], sve, sem)
      c2 = pltpu.async_copy(t_hbm.at[idx_so], svo, sem)
      c3 = pltpu.async_copy(t_hbm.at[idx_re], rve, sem)
      c4 = pltpu.async_copy(t_hbm.at[idx_ro], rvo, sem)
      c1.wait()
      c2.wait()
      c3.wait()
      c4.wait()
      pltpu.sync_copy(sve, g_hbm.at[pl.ds(base, CHUNK), pl.ds(0, TW)])
      pltpu.sync_copy(rve, g_hbm.at[pl.ds(base, CHUNK), pl.ds(TW, TW)])
      pltpu.sync_copy(svo, g_hbm.at[pl.ds(base, CHUNK), pl.ds(2 * TW, TW)])
      pltpu.sync_copy(rvo, g_hbm.at[pl.ds(base, CHUNK), pl.ds(3 * TW, TW)])

    return 0

  lax.fori_loop(0, K_STEPS, step, 0)


_sc_gather = functools.partial(
    pl.kernel,
    out_type=jax.ShapeDtypeStruct((N_PAIRS, 128), jnp.float32),
    mesh=plsc.VectorSubcoreMesh(core_axis_name="c", subcore_axis_name="s",
                                num_cores=NUM_CORES,
                                num_subcores=NUM_SUBCORES),
    scratch_types=[
        pltpu.VMEM((CHUNK,), jnp.int32),
        pltpu.VMEM((CHUNK,), jnp.int32),
        pltpu.VMEM((CHUNK,), jnp.int32),
        pltpu.VMEM((CHUNK,), jnp.int32),
        pltpu.VMEM((CHUNK, TW), jnp.float32),
        pltpu.VMEM((CHUNK, TW), jnp.float32),
        pltpu.VMEM((CHUNK, TW), jnp.float32),
        pltpu.VMEM((CHUNK, TW), jnp.float32),
        pltpu.SemaphoreType.DMA,
    ],
    compiler_params=pltpu.CompilerParams(use_tc_tiling_on_sc=False),
)(_sc_gather_body)


def _unpack_pos(col):
  hi = jnp.uint32(0xFFFF0000)
  u = lax.bitcast_convert_type(col, jnp.uint32)
  px = lax.bitcast_convert_type(u & hi, jnp.float32)
  py = lax.bitcast_convert_type(u << 16, jnp.float32)
  return px, py


def _wrap(d):
  half = BOX / 2
  d = jnp.where(d > half, d - BOX, d)
  return jnp.where(d <= -half, d + BOX, d)


def _tc_mlp_body(g_ref, m_ref, b1_ref, w2_ref, b2_ref, o_ref):
  g = g_ref[...]
  sxe, sye = _unpack_pos(g[:, TW - 1:TW])
  rxe, rye = _unpack_pos(g[:, 2 * TW - 1:2 * TW])
  sxo, syo = _unpack_pos(g[:, 3 * TW - 1:3 * TW])
  rxo, ryo = _unpack_pos(g[:, 4 * TW - 1:4 * TW])
  d0e = _wrap(sxe - rxe)
  d1e = _wrap(sye - rye)
  d0o = _wrap(sxo - rxo)
  d1o = _wrap(syo - ryo)
  lane = lax.broadcasted_iota(jnp.int32, (BEH, 128), 1)
  g = jnp.where(lane == TW - 1, d0e, g)
  g = jnp.where(lane == 2 * TW - 1, d1e, g)
  g = jnp.where(lane == 3 * TW - 1, d0o, g)
  g = jnp.where(lane == 4 * TW - 1, d1o, g)
  h = jnp.dot(g, m_ref[...], preferred_element_type=jnp.float32)
  h = jnp.maximum(h + b1_ref[...], 0.0)
  o = jnp.dot(h, w2_ref[...], preferred_element_type=jnp.float32)
  o = jnp.maximum(o + b2_ref[...], 0.0)
  o_ref[0] = o[:, 0:D_HID]
  o_ref[1] = o[:, D_HID:2 * D_HID]


def _tc_mlp(g, m, b1r, w2, b2r):
  return pl.pallas_call(
      _tc_mlp_body,
      grid=(TC_GRID,),
      in_specs=[
          pl.BlockSpec((BEH, 128), lambda i: (i, 0)),
          pl.BlockSpec((128, 128), lambda i: (0, 0)),
          pl.BlockSpec((1, 128), lambda i: (0, 0)),
          pl.BlockSpec((128, 128), lambda i: (0, 0)),
          pl.BlockSpec((1, 128), lambda i: (0, 0)),
      ],
      out_specs=pl.BlockSpec((2, BEH, D_HID), lambda i: (0, i, 0)),
      out_shape=jax.ShapeDtypeStruct((2, N_PAIRS, D_HID), jnp.float32),
  )(g, m, b1r, w2, b2r)


@jax.jit
def kernel(V_no_pos, V_pos, R_s, R_r, W1, b1, W2, b2):
  v = V_no_pos[0]
  p = V_pos[0]
  px_bits = lax.bitcast_convert_type(
      p[:, 0].astype(jnp.bfloat16), jnp.uint16).astype(jnp.uint32)
  py_bits = lax.bitcast_convert_type(
      p[:, 1].astype(jnp.bfloat16), jnp.uint16).astype(jnp.uint32)
  packed = lax.bitcast_convert_type((px_bits << 16) | py_bits, jnp.float32)
  table = jnp.concatenate([v, packed[:, None]], axis=1)

  rs = R_s.reshape(N_EDGES).astype(jnp.int32)
  rr = R_r.reshape(N_EDGES).astype(jnp.int32)
  g = _sc_gather(table, rs, rr)

  mhalf = jnp.concatenate(
      [W1[:, 0:D_NODE].T,
       W1[:, 2 * D_NODE:2 * D_NODE + 1].T,
       W1[:, D_NODE:2 * D_NODE].T,
       W1[:, 2 * D_NODE + 1:2 * D_NODE + 2].T],
      axis=0)
  z = jnp.zeros((64, 64), jnp.float32)
  m = jnp.concatenate(
      [jnp.concatenate([mhalf, z], axis=1),
       jnp.concatenate([z, mhalf], axis=1)], axis=0)
  w2 = jnp.concatenate(
      [jnp.concatenate([W2.T, z], axis=1),
       jnp.concatenate([z, W2.T], axis=1)], axis=0)
  b1r = jnp.concatenate([b1, b1])[None, :]
  b2r = jnp.concatenate([b2, b2])[None, :]

  out = _tc_mlp(g, m, b1r, w2, b2r)
  return out.reshape(1, N_EDGES, D_HID)
